# Initial kernel scaffold; baseline (speedup 1.0000x reference)
import jax
import jax.numpy as jnp
from jax.experimental import pallas as pl


def kernel(x, edge_index, edge_weight, W_xz_0, W_xz_1, b_xz, W_hz_0, W_hz_1, b_hz, W_xr_0, W_xr_1, b_xr, W_hr_0, W_hr_1, b_hr, W_xh_0, W_xh_1, b_xh, W_hh_0, W_hh_1, b_hh, lin_W, lin_b):
    raise NotImplementedError("write your pallas kernel here")


# speedup vs baseline: 17.1870x; 17.1870x over previous
import jax
import jax.numpy as jnp
from jax import lax
from jax.experimental import pallas as pl
from jax.experimental.pallas import tpu as pltpu
from jax.experimental.pallas import tpu_sc as plsc

N = 10000
E = 320000
F = 128
FH = F // 2
NC = 2
NS = 16
L = 16
NPAD = 10240
C = 80
CHUNKS = E // C
CPB = CHUNKS // (NC * NS)
SROWS = NPAD // NS
ZROWS = 32

_RSQRT_MAGIC = 0x5F3759DF


def _newton_rsqrt16(d):
    i = lax.bitcast_convert_type(d, jnp.int32)
    y = lax.bitcast_convert_type(_RSQRT_MAGIC - lax.shift_right_logical(i, 1),
                                 jnp.float32)
    half_d = 0.5 * d
    for _ in range(3):
        y = y * (1.5 - half_d * y * y)
    return jnp.where(d > 0.0, y, 0.0)


def _sc_aggregate(xh_hbm, row_h, col_h, w_h, s_out, deg_out,
                  zbuf, zdeg, rowc, colc, wc, dinv_v, rows_buf,
                  s_sh, deg_sh, sem):
    cid = lax.axis_index("c")
    sid = lax.axis_index("s")

    def zero16(i, _):
        zbuf[lax.div(i, FH // L), pl.ds(lax.rem(i, FH // L) * L, L)] = (
            jnp.zeros((L,), jnp.float32))
        return 0
    lax.fori_loop(0, ZROWS * (FH // L), zero16, 0)

    def zdeg16(i, _):
        zdeg[pl.ds(i * L, L)] = jnp.zeros((L,), jnp.float32)
        return 0
    lax.fori_loop(0, SROWS // L, zdeg16, 0)

    pltpu.sync_copy(zdeg, deg_sh.at[pl.ds(sid * SROWS, SROWS)])
    for m in range(SROWS // ZROWS):
        pltpu.sync_copy(zbuf, s_sh.at[pl.ds(sid * SROWS + m * ZROWS, ZROWS), :])
    plsc.subcore_barrier()

    def deg_half(h, _):
        blk = sid * 2 + h
        pltpu.sync_copy(row_h.at[blk], rowc)
        pltpu.sync_copy(col_h.at[blk], colc)
        pltpu.sync_copy(w_h.at[blk], wc)

        def wprime(i, _):
            j = lax.div(i, C // L)
            k = lax.rem(i, C // L) * L
            r16 = rowc[j, pl.ds(k, L)]
            c16 = colc[j, pl.ds(k, L)]
            w16 = wc[j, pl.ds(k, L)]
            wc[j, pl.ds(k, L)] = jnp.where(r16 == c16, 0.0, w16)
            return 0
        lax.fori_loop(0, CPB * (C // L), wprime, 0)

        def scat(j, _):
            pltpu.sync_copy(wc.at[j], deg_sh.at[rowc.at[j]], add=True)
            return 0
        lax.fori_loop(0, CPB, scat, 0)
        return 0
    lax.fori_loop(0, 2, deg_half, 0)
    plsc.subcore_barrier()

    pltpu.sync_copy(deg_sh, dinv_v)

    @pl.when(cid == 0)
    def _():
        pltpu.sync_copy(deg_sh.at[pl.ds(sid * SROWS, SROWS)],
                        deg_out.at[pl.ds(sid * SROWS, SROWS)])

    def newton(i, _):
        d = dinv_v[pl.ds(i * L, L)]
        dinv_v[pl.ds(i * L, L)] = _newton_rsqrt16(d)
        return 0
    lax.fori_loop(0, NPAD // L, newton, 0)

    def agg_half(h, _):
        blk = sid * 2 + h
        pltpu.sync_copy(row_h.at[blk], rowc)
        pltpu.sync_copy(col_h.at[blk], colc)
        pltpu.sync_copy(w_h.at[blk], wc)

        def ucalc(i, _):
            j = lax.div(i, C // L)
            k = lax.rem(i, C // L) * L
            r16 = rowc[j, pl.ds(k, L)]
            c16 = colc[j, pl.ds(k, L)]
            w16 = wc[j, pl.ds(k, L)]
            dv16 = plsc.load_gather(dinv_v, [r16])
            wc[j, pl.ds(k, L)] = jnp.where(r16 == c16, 0.0, w16) * dv16
            return 0
        lax.fori_loop(0, CPB * (C // L), ucalc, 0)

        def chunk(j, _):
            pltpu.async_copy(xh_hbm.at[cid].at[rowc.at[j]], rows_buf, sem).wait()

            def scale_edge(e, _):
                sel = jnp.zeros((L,), jnp.int32)
                u16 = plsc.load_gather(wc, [sel + j, sel + e])
                for q in range(FH // L):
                    rows_buf[e, pl.ds(q * L, L)] = (
                        rows_buf[e, pl.ds(q * L, L)] * u16)
                return 0
            lax.fori_loop(0, C, scale_edge, 0)

            pltpu.sync_copy(rows_buf, s_sh.at[colc.at[j]], add=True)
            return 0
        lax.fori_loop(0, CPB, chunk, 0)
        return 0
    lax.fori_loop(0, 2, agg_half, 0)
    plsc.subcore_barrier()

    pltpu.sync_copy(s_sh.at[pl.ds(sid * SROWS, SROWS), :],
                    s_out.at[cid, pl.ds(sid * SROWS, SROWS), :])


def _run_sc(xh, row_h, col_h, w_h):
    mesh = plsc.VectorSubcoreMesh(core_axis_name="c", subcore_axis_name="s")
    return pl.kernel(
        _sc_aggregate,
        out_type=(
            jax.ShapeDtypeStruct((NC, NPAD, FH), jnp.float32),
            jax.ShapeDtypeStruct((NPAD,), jnp.float32),
        ),
        mesh=mesh,
        compiler_params=pltpu.CompilerParams(needs_layout_passes=False,
                                             use_tc_tiling_on_sc=False),
        scratch_types=[
            pltpu.VMEM((ZROWS, FH), jnp.float32),
            pltpu.VMEM((SROWS,), jnp.float32),
            pltpu.VMEM((CPB, C), jnp.int32),
            pltpu.VMEM((CPB, C), jnp.int32),
            pltpu.VMEM((CPB, C), jnp.float32),
            pltpu.VMEM((NPAD,), jnp.float32),
            pltpu.VMEM((C, FH), jnp.float32),
            pltpu.VMEM_SHARED((NPAD, FH), jnp.float32),
            pltpu.VMEM_SHARED((NPAD,), jnp.float32),
            pltpu.SemaphoreType.DMA,
        ],
    )(xh, row_h, col_h, w_h)


def _tc_dense(x_ref, sa_ref, sb_ref, deg_ref, wx_ref, wta_ref, wtb_ref,
              bias_ref, lin_ref, lb_ref, out_ref):
    deg = deg_ref[...]
    dinv = jnp.where(deg > 0.0, lax.rsqrt(deg), 0.0)
    ab = (jnp.dot(x_ref[...], wx_ref[...], preferred_element_type=jnp.float32)
          + jnp.dot(sa_ref[...] * (-dinv), wta_ref[...],
                    preferred_element_type=jnp.float32)
          + jnp.dot(sb_ref[...] * (-dinv), wtb_ref[...],
                    preferred_element_type=jnp.float32)
          + bias_ref[...])
    z = jax.nn.sigmoid(ab[:, :F])
    t = jnp.tanh(ab[:, F:])
    h = jnp.maximum((1.0 - z) * t, 0.0)
    out_ref[...] = (jnp.dot(h, lin_ref[...], preferred_element_type=jnp.float32)
                    + lb_ref[...])


def _run_tc(x, sa, sb, deg, wx, wta, wtb, bias, lin_w, lin_b):
    blk = 1000
    grid = (N // blk,)
    return pl.pallas_call(
        _tc_dense,
        grid=grid,
        in_specs=[
            pl.BlockSpec((blk, F), lambda i: (i, 0)),
            pl.BlockSpec((blk, FH), lambda i: (i, 0)),
            pl.BlockSpec((blk, FH), lambda i: (i, 0)),
            pl.BlockSpec((blk, 1), lambda i: (i, 0)),
            pl.BlockSpec((F, 2 * F), lambda i: (0, 0)),
            pl.BlockSpec((FH, 2 * F), lambda i: (0, 0)),
            pl.BlockSpec((FH, 2 * F), lambda i: (0, 0)),
            pl.BlockSpec((1, 2 * F), lambda i: (0, 0)),
            pl.BlockSpec((F, F), lambda i: (0, 0)),
            pl.BlockSpec((1, F), lambda i: (0, 0)),
        ],
        out_specs=pl.BlockSpec((blk, F), lambda i: (i, 0)),
        out_shape=jax.ShapeDtypeStruct((N, F), jnp.float32),
    )(x, sa, sb, deg, wx, wta, wtb, bias, lin_w, lin_b)


@jax.jit
def kernel(x, edge_index, edge_weight, W_xz_0, W_xz_1, b_xz, W_hz_0, W_hz_1,
           b_hz, W_xr_0, W_xr_1, b_xr, W_hr_0, W_hr_1, b_hr, W_xh_0, W_xh_1,
           b_xh, W_hh_0, W_hh_1, b_hh, lin_W, lin_b):
    row_h = edge_index[0].reshape(NC * NS, CPB, C)
    col_h = edge_index[1].reshape(NC * NS, CPB, C)
    w_h = edge_weight.reshape(NC * NS, CPB, C)
    xh = x.reshape(N, NC, FH).transpose(1, 0, 2)

    s_half, deg = _run_sc(xh, row_h, col_h, w_h)

    wx = jnp.concatenate([W_xz_0, W_xh_0], axis=1)
    wt = jnp.concatenate([W_xz_1, W_xh_1], axis=1)
    bias = jnp.concatenate([b_xz + b_hz, b_xh + b_hh])[None, :]
    out = _run_tc(x, s_half[0, :N], s_half[1, :N], deg[:N].reshape(N, 1),
                  wx, wt[:FH], wt[FH:], bias, lin_W, lin_b[None, :])
    return out
